# baseline (speedup 1.0000x reference)
import functools

import jax
import jax.numpy as jnp
from jax import lax
from jax.experimental import pallas as pl
from jax.experimental.pallas import tpu as pltpu
from jax.experimental.pallas import tpu_sc as plsc

N = 10000
E = 320000
D = 128
NP = 10240
NC = 2
NS = 16
NW = NC * NS
CHUNK = 64
CPW = 160
NCH = NW * CPW
EPAD = NCH * CHUNK
RPT = NP // NS
DG = 128
IDXB = 40
NBUF = 4

def _deg_body(dstc_hbm, ones_hbm, zeros_hbm, out_hbm, dstb_v, ones_v, buf_v, deg_sh):
    c = lax.axis_index("c")
    s = lax.axis_index("s")
    wid = s * NC + c
    base = s * RPT

    pltpu.sync_copy(dstc_hbm.at[wid], dstb_v)
    pltpu.sync_copy(ones_hbm, ones_v)
    pltpu.sync_copy(zeros_hbm, buf_v)

    def zero_body(k, carry):
        pltpu.sync_copy(buf_v, deg_sh.at[pl.ds(base + k * 128, 128)])
        return carry

    lax.fori_loop(0, RPT // 128, zero_body, 0)
    plsc.subcore_barrier()

    def edge_body(j, carry):
        pltpu.sync_copy(ones_v, deg_sh.at[dstb_v.at[j]], add=True)
        return carry

    lax.fori_loop(0, CPW, edge_body, 0)
    plsc.subcore_barrier()

    def rb_body(k, carry):
        r = base + k * 128
        pltpu.sync_copy(deg_sh.at[pl.ds(r, 128)], buf_v)
        pltpu.sync_copy(buf_v, out_hbm.at[c, pl.ds(r, 128)])
        return carry

    lax.fori_loop(0, RPT // 128, rb_body, 0)


def _msg_body(y_hbm, srcc_hbm, dstc_hbm, out_hbm,
              srcb_v, dstb_v, rows0_v, rows1_v, rows2_v, rows3_v, acc_sh,
              sem0, sem1, sem2, sem3):
    c = lax.axis_index("c")
    s = lax.axis_index("s")
    wid = s * NC + c
    base = s * RPT
    rows = [rows0_v, rows1_v, rows2_v, rows3_v]
    sems = [sem0, sem1, sem2, sem3]

    def init_body(k, carry):
        r = base + k * CHUNK
        pltpu.sync_copy(y_hbm.at[pl.ds(r, CHUNK)], rows0_v)
        pltpu.sync_copy(rows0_v, acc_sh.at[pl.ds(r, CHUNK)])
        return carry

    lax.fori_loop(0, RPT // CHUNK, init_body, 0)
    plsc.subcore_barrier()

    def gather_start(j, b):
        pltpu.async_copy(y_hbm.at[srcb_v.at[j]], rows[b], sems[b])

    def gather_wait_scatter(j, b):
        pltpu.make_async_copy(y_hbm.at[srcb_v.at[j]], rows[b], sems[b]).wait()
        pltpu.sync_copy(rows[b], acc_sh.at[dstb_v.at[j]], add=True)

    def run_pass(hbm_off, n):
        assert n % NBUF == 0
        pltpu.sync_copy(srcc_hbm.at[wid, pl.ds(hbm_off, n)], srcb_v.at[pl.ds(0, n)])
        pltpu.sync_copy(dstc_hbm.at[wid, pl.ds(hbm_off, n)], dstb_v.at[pl.ds(0, n)])
        for b in range(NBUF):
            gather_start(b, b)

        def edge_body(g, carry):
            j = g * NBUF
            for b in range(NBUF):
                gather_wait_scatter(j + b, b)
                gather_start(j + NBUF + b, b)
            return carry

        lax.fori_loop(0, n // NBUF - 1, edge_body, 0)
        for b in range(NBUF):
            gather_wait_scatter(n - NBUF + b, b)

    for po in range(0, CPW, IDXB):
        run_pass(po, IDXB)
    plsc.subcore_barrier()

    def rb_body(k, carry):
        r = base + k * CHUNK
        pltpu.sync_copy(acc_sh.at[pl.ds(r, CHUNK)], rows0_v)
        pltpu.sync_copy(rows0_v, out_hbm.at[c, pl.ds(r, CHUNK)])
        return carry

    lax.fori_loop(0, RPT // CHUNK, rb_body, 0)


def _tc1_body(x_ref, w1_ref, degp_ref, y_ref, dinv_ref):
    deg = degp_ref[0, 0:N, 0:1] + degp_ref[1, 0:N, 0:1] + 1.0
    dinv = lax.rsqrt(deg)
    xw = jnp.dot(x_ref[...], w1_ref[...], preferred_element_type=jnp.float32)
    y_ref[0:N, :] = xw * dinv
    y_ref[N:NP, :] = jnp.zeros((NP - N, D), jnp.float32)
    dinv_ref[...] = dinv


_tc1 = pl.pallas_call(
    _tc1_body,
    out_shape=(
        jax.ShapeDtypeStruct((NP, D), jnp.float32),
        jax.ShapeDtypeStruct((N, 1), jnp.float32),
    ),
)


def _tc2_body(accp_ref, y_ref, dinv_ref, b_ref, w2_ref, y2_ref):
    dinv = dinv_ref[...]
    t = accp_ref[0, 0:N, :] + accp_ref[1, 0:N, :] - y_ref[0:N, :]
    t = t * dinv + b_ref[...]
    h = jnp.where(t > 0, t, jnp.exp(t) - 1.0)
    y2 = jnp.dot(h, w2_ref[...], preferred_element_type=jnp.float32) * dinv
    y2_ref[0:N, :] = y2
    y2_ref[N:NP, :] = jnp.zeros((NP - N, D), jnp.float32)


_tc2 = pl.pallas_call(
    _tc2_body,
    out_shape=jax.ShapeDtypeStruct((NP, D), jnp.float32),
)


def _tc3_body(accp_ref, y2_ref, dinv_ref, b_ref, wl_ref, bl_ref, o_ref):
    dinv = dinv_ref[...]
    t = accp_ref[0, 0:N, :] + accp_ref[1, 0:N, :] - y2_ref[0:N, :]
    t = t * dinv + b_ref[...]
    h = jnp.where(t > 0, t, jnp.exp(t) - 1.0)
    o_ref[...] = jnp.dot(h, wl_ref[...], preferred_element_type=jnp.float32) + bl_ref[...]


_tc3 = pl.pallas_call(
    _tc3_body,
    out_shape=jax.ShapeDtypeStruct((N, 1), jnp.float32),
)


@functools.lru_cache(maxsize=1)
def _sc_kernels():
    mesh = plsc.VectorSubcoreMesh(
        core_axis_name="c", subcore_axis_name="s", num_cores=NC)
    deg_k = pl.kernel(
        _deg_body,
        mesh=mesh,
        out_type=jax.ShapeDtypeStruct((NC, NP, DG), jnp.float32),
        scratch_types=[
            pltpu.VMEM((CPW, CHUNK), jnp.int32),
            pltpu.VMEM((CHUNK, DG), jnp.float32),
            pltpu.VMEM((128, DG), jnp.float32),
            pltpu.VMEM_SHARED((NP, DG), jnp.float32),
        ],
    )
    msg_k = pl.kernel(
        _msg_body,
        mesh=mesh,
        out_type=jax.ShapeDtypeStruct((NC, NP, D), jnp.float32),
        scratch_types=[
            pltpu.VMEM((IDXB, CHUNK), jnp.int32),
            pltpu.VMEM((IDXB, CHUNK), jnp.int32),
            pltpu.VMEM((CHUNK, D), jnp.float32),
            pltpu.VMEM((CHUNK, D), jnp.float32),
            pltpu.VMEM((CHUNK, D), jnp.float32),
            pltpu.VMEM((CHUNK, D), jnp.float32),
            pltpu.VMEM_SHARED((NP, D), jnp.float32),
            pltpu.SemaphoreType.DMA,
            pltpu.SemaphoreType.DMA,
            pltpu.SemaphoreType.DMA,
            pltpu.SemaphoreType.DMA,
        ],
    )
    return deg_k, msg_k


def kernel(x, edge_index, W1, b1, W2, b2, Wl, bl):
    _deg_kernel, _msg_kernel = _sc_kernels()
    src = edge_index[0]
    dst = edge_index[1]
    pad = jnp.full((EPAD - E,), N, jnp.int32)
    srcc = jnp.concatenate([src, pad]).reshape(NW, CPW, CHUNK)
    dstc = jnp.concatenate([dst, pad]).reshape(NW, CPW, CHUNK)

    degp = _deg_kernel(dstc, jnp.ones((CHUNK, DG), jnp.float32),
                       jnp.zeros((128, DG), jnp.float32))
    y1, dinv = _tc1(x, W1, degp)
    accp1 = _msg_kernel(y1, srcc, dstc)
    y2 = _tc2(accp1, y1, dinv, b1.reshape(1, D), W2)
    accp2 = _msg_kernel(y2, srcc, dstc)
    out = _tc3(accp2, y2, dinv, b2.reshape(1, D), Wl, bl.reshape(1, 1))
    return out


# speedup vs baseline: 1.8019x; 1.8019x over previous
import functools

import jax
import jax.numpy as jnp
from jax import lax
from jax.experimental import pallas as pl
from jax.experimental.pallas import tpu as pltpu
from jax.experimental.pallas import tpu_sc as plsc

N = 10000
E = 320000
D = 128
NP = 10240
NC = 2
NS = 16
NW = NC * NS
CHUNK = 128
CPW = 79
NCH = NW * CPW
EPAD = NCH * CHUNK
RPT = NP // NS
DG = 128
IDXB = 40

def _deg_body(dstc_hbm, const_hbm, out_hbm, dstb_v, ones_v, buf_v, deg_sh):
    c = lax.axis_index("c")
    s = lax.axis_index("s")
    wid = s * NC + c
    base = s * RPT

    pltpu.sync_copy(dstc_hbm.at[wid], dstb_v)
    pltpu.sync_copy(const_hbm.at[0], ones_v)
    pltpu.sync_copy(const_hbm.at[1], buf_v)

    def zero_body(k, carry):
        pltpu.sync_copy(buf_v, deg_sh.at[pl.ds(base + k * CHUNK, CHUNK)])
        return carry

    lax.fori_loop(0, RPT // CHUNK, zero_body, 0)
    plsc.subcore_barrier()

    def edge_body(j, carry):
        pltpu.sync_copy(ones_v, deg_sh.at[dstb_v.at[j]], add=True)
        return carry

    lax.fori_loop(0, CPW, edge_body, 0)
    plsc.subcore_barrier()

    def rb_body(k, carry):
        r = base + k * CHUNK
        pltpu.sync_copy(deg_sh.at[pl.ds(r, CHUNK)], buf_v)
        pltpu.sync_copy(buf_v, out_hbm.at[c, pl.ds(r, CHUNK)])
        return carry

    lax.fori_loop(0, RPT // CHUNK, rb_body, 0)


def _msg_body(y_hbm, srcc_hbm, dstc_hbm, out_hbm,
              srcb_v, dstb_v, rows0_v, rows1_v, acc_sh, sem0, sem1):
    c = lax.axis_index("c")
    s = lax.axis_index("s")
    wid = s * NC + c
    base = s * RPT

    ibuf = [rows0_v, rows1_v]
    isem = [sem0, sem1]
    nin = RPT // CHUNK
    pltpu.async_copy(y_hbm.at[pl.ds(base, CHUNK)], rows0_v, sem0)
    for k in range(nin):
        r = base + k * CHUNK
        b = k % 2
        pltpu.make_async_copy(y_hbm.at[pl.ds(r, CHUNK)], ibuf[b], isem[b]).wait()
        if k + 1 < nin:
            pltpu.async_copy(y_hbm.at[pl.ds(r + CHUNK, CHUNK)],
                             ibuf[1 - b], isem[1 - b])
        pltpu.sync_copy(ibuf[b], acc_sh.at[pl.ds(r, CHUNK)])
    plsc.subcore_barrier()

    def gather_start(j, rows_v, sem):
        pltpu.async_copy(y_hbm.at[srcb_v.at[j]], rows_v, sem)

    def gather_wait_scatter(j, rows_v, sem):
        pltpu.make_async_copy(y_hbm.at[srcb_v.at[j]], rows_v, sem).wait()
        pltpu.sync_copy(rows_v, acc_sh.at[dstb_v.at[j]], add=True)

    def run_pass(hbm_off, n):
        pltpu.sync_copy(srcc_hbm.at[wid, pl.ds(hbm_off, n)], srcb_v.at[pl.ds(0, n)])
        pltpu.sync_copy(dstc_hbm.at[wid, pl.ds(hbm_off, n)], dstb_v.at[pl.ds(0, n)])
        gather_start(0, rows0_v, sem0)

        def edge_body(k, carry):
            j = 2 * k
            gather_start(j + 1, rows1_v, sem1)
            gather_wait_scatter(j, rows0_v, sem0)
            gather_start(j + 2, rows0_v, sem0)
            gather_wait_scatter(j + 1, rows1_v, sem1)
            return carry

        m = (n - 1) // 2
        lax.fori_loop(0, m, edge_body, 0)
        if n % 2:
            gather_wait_scatter(n - 1, rows0_v, sem0)
        else:
            gather_start(n - 1, rows1_v, sem1)
            gather_wait_scatter(n - 2, rows0_v, sem0)
            gather_wait_scatter(n - 1, rows1_v, sem1)

    off = 0
    left = CPW
    while left > 0:
        nn = min(IDXB, left)
        run_pass(off, nn)
        off += nn
        left -= nn

    plsc.subcore_barrier()

    pltpu.async_copy(acc_sh.at[pl.ds(base, CHUNK)], rows0_v, sem0)
    for k in range(RPT // CHUNK):
        r = base + k * CHUNK
        b = k % 2
        pltpu.make_async_copy(acc_sh.at[pl.ds(r, CHUNK)], ibuf[b], isem[b]).wait()
        if k + 1 < RPT // CHUNK:
            pltpu.async_copy(acc_sh.at[pl.ds(r + CHUNK, CHUNK)],
                             ibuf[1 - b], isem[1 - b])
        pltpu.sync_copy(ibuf[b], out_hbm.at[c, pl.ds(r, CHUNK)])


def _tcmm_body(x_ref, w1_ref, xw_ref):
    xw_ref[...] = jnp.dot(x_ref[...], w1_ref[...],
                          preferred_element_type=jnp.float32)


_tcmm = pl.pallas_call(
    _tcmm_body,
    out_shape=jax.ShapeDtypeStruct((N, D), jnp.float32),
)


def _tc1_body(xw_ref, degp_ref, y_ref, dinv_ref):
    deg = degp_ref[0, 0:N, 0:1] + degp_ref[1, 0:N, 0:1] + 1.0
    dinv = lax.rsqrt(deg)
    y_ref[0:N, :] = xw_ref[...] * dinv
    y_ref[N:NP, :] = jnp.zeros((NP - N, D), jnp.float32)
    dinv_ref[...] = dinv


_tc1 = pl.pallas_call(
    _tc1_body,
    out_shape=(
        jax.ShapeDtypeStruct((NP, D), jnp.float32),
        jax.ShapeDtypeStruct((N, 1), jnp.float32),
    ),
)


def _tc2_body(accp_ref, y_ref, dinv_ref, b_ref, w2_ref, y2_ref):
    dinv = dinv_ref[...]
    t = accp_ref[0, 0:N, :] + accp_ref[1, 0:N, :] - y_ref[0:N, :]
    t = t * dinv + b_ref[...]
    h = jnp.where(t > 0, t, jnp.exp(t) - 1.0)
    y2 = jnp.dot(h, w2_ref[...], preferred_element_type=jnp.float32) * dinv
    y2_ref[0:N, :] = y2
    y2_ref[N:NP, :] = jnp.zeros((NP - N, D), jnp.float32)


_tc2 = pl.pallas_call(
    _tc2_body,
    out_shape=jax.ShapeDtypeStruct((NP, D), jnp.float32),
)


def _tc3_body(accp_ref, y2_ref, dinv_ref, b_ref, wl_ref, bl_ref, o_ref):
    dinv = dinv_ref[...]
    t = accp_ref[0, 0:N, :] + accp_ref[1, 0:N, :] - y2_ref[0:N, :]
    t = t * dinv + b_ref[...]
    h = jnp.where(t > 0, t, jnp.exp(t) - 1.0)
    o_ref[...] = jnp.dot(h, wl_ref[...], preferred_element_type=jnp.float32) + bl_ref[...]


_tc3 = pl.pallas_call(
    _tc3_body,
    out_shape=jax.ShapeDtypeStruct((N, 1), jnp.float32),
)


@functools.lru_cache(maxsize=1)
def _sc_kernels():
    mesh = plsc.VectorSubcoreMesh(
        core_axis_name="c", subcore_axis_name="s", num_cores=NC)
    deg_k = pl.kernel(
        _deg_body,
        mesh=mesh,
        out_type=jax.ShapeDtypeStruct((NC, NP, DG), jnp.float32),
        scratch_types=[
            pltpu.VMEM((CPW, CHUNK), jnp.int32),
            pltpu.VMEM((CHUNK, DG), jnp.float32),
            pltpu.VMEM((CHUNK, DG), jnp.float32),
            pltpu.VMEM_SHARED((NP, DG), jnp.float32),
        ],
    )
    msg_k = pl.kernel(
        _msg_body,
        mesh=mesh,
        out_type=jax.ShapeDtypeStruct((NC, NP, D), jnp.float32),
        scratch_types=[
            pltpu.VMEM((IDXB, CHUNK), jnp.int32),
            pltpu.VMEM((IDXB, CHUNK), jnp.int32),
            pltpu.VMEM((CHUNK, D), jnp.float32),
            pltpu.VMEM((CHUNK, D), jnp.float32),
            pltpu.VMEM_SHARED((NP, D), jnp.float32),
            pltpu.SemaphoreType.DMA,
            pltpu.SemaphoreType.DMA,
        ],
    )
    return deg_k, msg_k


def kernel(x, edge_index, W1, b1, W2, b2, Wl, bl):
    _deg_kernel, _msg_kernel = _sc_kernels()
    src = edge_index[0]
    dst = edge_index[1]
    pad = jnp.full((EPAD - E,), N, jnp.int32)
    srcc = jnp.concatenate([src, pad]).reshape(NW, CPW, CHUNK)
    dstc = jnp.concatenate([dst, pad]).reshape(NW, CPW, CHUNK)

    const = jnp.stack([jnp.ones((CHUNK, DG), jnp.float32),
                       jnp.zeros((CHUNK, DG), jnp.float32)])
    xw = _tcmm(x, W1)
    degp = _deg_kernel(dstc, const)
    y1, dinv = _tc1(xw, degp)
    accp1 = _msg_kernel(y1, srcc, dstc)
    y2 = _tc2(accp1, y1, dinv, b1.reshape(1, D), W2)
    accp2 = _msg_kernel(y2, srcc, dstc)
    out = _tc3(accp2, y2, dinv, b2.reshape(1, D), Wl, bl.reshape(1, 1))
    return out


# speedup vs baseline: 1.8110x; 1.0050x over previous
import functools

import jax
import jax.numpy as jnp
from jax import lax
from jax.experimental import pallas as pl
from jax.experimental.pallas import tpu as pltpu
from jax.experimental.pallas import tpu_sc as plsc

N = 10000
E = 320000
D = 128
NP = 10240
NC = 2
NS = 16
NW = NC * NS
CHUNK = 128
CPW = 79
NCH = NW * CPW
EPAD = NCH * CHUNK
RPT = NP // NS
DG = 128
IDXB = 40

DEGQ = 8


def _deg_body(dstc_hbm, const_hbm, out_hbm, dstb_v, ones_v, buf_v, deg_sh,
              semq, sem0, sem1):
    c = lax.axis_index("c")
    s = lax.axis_index("s")
    wid = s * NC + c
    base = s * RPT

    pltpu.sync_copy(dstc_hbm.at[wid], dstb_v)
    pltpu.sync_copy(const_hbm.at[0], ones_v)
    pltpu.sync_copy(const_hbm.at[1], buf_v)

    def zero_body(k, carry):
        pltpu.sync_copy(buf_v, deg_sh.at[pl.ds(base + k * CHUNK, CHUNK)])
        return carry

    lax.fori_loop(0, RPT // CHUNK, zero_body, 0)
    plsc.subcore_barrier()

    def sc_start(j):
        pltpu.async_copy(ones_v, deg_sh.at[dstb_v.at[j]], semq, add=True)

    def sc_wait(j):
        pltpu.make_async_copy(ones_v, deg_sh.at[dstb_v.at[j]], semq).wait()

    for j in range(DEGQ):
        sc_start(j)

    def edge_body(j, carry):
        sc_wait(j)
        sc_start(j + DEGQ)
        return carry

    lax.fori_loop(0, CPW - DEGQ, edge_body, 0)

    def drain_body(j, carry):
        sc_wait(j)
        return carry

    lax.fori_loop(CPW - DEGQ, CPW, drain_body, 0)
    plsc.subcore_barrier()

    ibuf = [buf_v, ones_v]
    isem = [sem0, sem1]
    nrb = RPT // CHUNK
    pltpu.async_copy(deg_sh.at[pl.ds(base, CHUNK)], buf_v, sem0)
    for k in range(nrb):
        r = base + k * CHUNK
        b = k % 2
        pltpu.make_async_copy(deg_sh.at[pl.ds(r, CHUNK)], ibuf[b], isem[b]).wait()
        if k + 1 < nrb:
            pltpu.async_copy(deg_sh.at[pl.ds(r + CHUNK, CHUNK)],
                             ibuf[1 - b], isem[1 - b])
        pltpu.sync_copy(ibuf[b], out_hbm.at[c, pl.ds(r, CHUNK)])


def _msg_body(y_hbm, srcc_hbm, dstc_hbm, out_hbm,
              srcb_v, dstb_v, rows0_v, rows1_v, acc_sh, sem0, sem1):
    c = lax.axis_index("c")
    s = lax.axis_index("s")
    wid = s * NC + c
    base = s * RPT

    ibuf = [rows0_v, rows1_v]
    isem = [sem0, sem1]
    nin = RPT // CHUNK
    pltpu.async_copy(y_hbm.at[pl.ds(base, CHUNK)], rows0_v, sem0)
    for k in range(nin):
        r = base + k * CHUNK
        b = k % 2
        pltpu.make_async_copy(y_hbm.at[pl.ds(r, CHUNK)], ibuf[b], isem[b]).wait()
        if k + 1 < nin:
            pltpu.async_copy(y_hbm.at[pl.ds(r + CHUNK, CHUNK)],
                             ibuf[1 - b], isem[1 - b])
        pltpu.sync_copy(ibuf[b], acc_sh.at[pl.ds(r, CHUNK)])
    plsc.subcore_barrier()

    def gather_start(j, rows_v, sem):
        pltpu.async_copy(y_hbm.at[srcb_v.at[j]], rows_v, sem)

    def gather_wait_scatter(j, rows_v, sem):
        pltpu.make_async_copy(y_hbm.at[srcb_v.at[j]], rows_v, sem).wait()
        pltpu.sync_copy(rows_v, acc_sh.at[dstb_v.at[j]], add=True)

    def run_pass(hbm_off, n):
        pltpu.sync_copy(srcc_hbm.at[wid, pl.ds(hbm_off, n)], srcb_v.at[pl.ds(0, n)])
        pltpu.sync_copy(dstc_hbm.at[wid, pl.ds(hbm_off, n)], dstb_v.at[pl.ds(0, n)])
        gather_start(0, rows0_v, sem0)

        def edge_body(k, carry):
            j = 2 * k
            gather_start(j + 1, rows1_v, sem1)
            gather_wait_scatter(j, rows0_v, sem0)
            gather_start(j + 2, rows0_v, sem0)
            gather_wait_scatter(j + 1, rows1_v, sem1)
            return carry

        m = (n - 1) // 2
        lax.fori_loop(0, m, edge_body, 0)
        if n % 2:
            gather_wait_scatter(n - 1, rows0_v, sem0)
        else:
            gather_start(n - 1, rows1_v, sem1)
            gather_wait_scatter(n - 2, rows0_v, sem0)
            gather_wait_scatter(n - 1, rows1_v, sem1)

    off = 0
    left = CPW
    while left > 0:
        nn = min(IDXB, left)
        run_pass(off, nn)
        off += nn
        left -= nn

    plsc.subcore_barrier()

    pltpu.async_copy(acc_sh.at[pl.ds(base, CHUNK)], rows0_v, sem0)
    for k in range(RPT // CHUNK):
        r = base + k * CHUNK
        b = k % 2
        pltpu.make_async_copy(acc_sh.at[pl.ds(r, CHUNK)], ibuf[b], isem[b]).wait()
        if k + 1 < RPT // CHUNK:
            pltpu.async_copy(acc_sh.at[pl.ds(r + CHUNK, CHUNK)],
                             ibuf[1 - b], isem[1 - b])
        pltpu.sync_copy(ibuf[b], out_hbm.at[c, pl.ds(r, CHUNK)])


def _tcmm_body(x_ref, w1_ref, xw_ref):
    xw_ref[...] = jnp.dot(x_ref[...], w1_ref[...],
                          preferred_element_type=jnp.float32)


_tcmm = pl.pallas_call(
    _tcmm_body,
    out_shape=jax.ShapeDtypeStruct((N, D), jnp.float32),
)


def _tc1_body(xw_ref, degp_ref, y_ref, dinv_ref):
    deg = degp_ref[0, 0:N, 0:1] + degp_ref[1, 0:N, 0:1] + 1.0
    dinv = lax.rsqrt(deg)
    y_ref[0:N, :] = xw_ref[...] * dinv
    y_ref[N:NP, :] = jnp.zeros((NP - N, D), jnp.float32)
    dinv_ref[...] = dinv


_tc1 = pl.pallas_call(
    _tc1_body,
    out_shape=(
        jax.ShapeDtypeStruct((NP, D), jnp.float32),
        jax.ShapeDtypeStruct((N, 1), jnp.float32),
    ),
)


def _tc2_body(accp_ref, y_ref, dinv_ref, b_ref, w2_ref, y2_ref):
    dinv = dinv_ref[...]
    t = accp_ref[0, 0:N, :] + accp_ref[1, 0:N, :] - y_ref[0:N, :]
    t = t * dinv + b_ref[...]
    h = jnp.where(t > 0, t, jnp.exp(t) - 1.0)
    y2 = jnp.dot(h, w2_ref[...], preferred_element_type=jnp.float32) * dinv
    y2_ref[0:N, :] = y2
    y2_ref[N:NP, :] = jnp.zeros((NP - N, D), jnp.float32)


_tc2 = pl.pallas_call(
    _tc2_body,
    out_shape=jax.ShapeDtypeStruct((NP, D), jnp.float32),
)


def _tc3_body(accp_ref, y2_ref, dinv_ref, b_ref, wl_ref, bl_ref, o_ref):
    dinv = dinv_ref[...]
    t = accp_ref[0, 0:N, :] + accp_ref[1, 0:N, :] - y2_ref[0:N, :]
    t = t * dinv + b_ref[...]
    h = jnp.where(t > 0, t, jnp.exp(t) - 1.0)
    o_ref[...] = jnp.dot(h, wl_ref[...], preferred_element_type=jnp.float32) + bl_ref[...]


_tc3 = pl.pallas_call(
    _tc3_body,
    out_shape=jax.ShapeDtypeStruct((N, 1), jnp.float32),
)


@functools.lru_cache(maxsize=1)
def _sc_kernels():
    mesh = plsc.VectorSubcoreMesh(
        core_axis_name="c", subcore_axis_name="s", num_cores=NC)
    deg_k = pl.kernel(
        _deg_body,
        mesh=mesh,
        out_type=jax.ShapeDtypeStruct((NC, NP, DG), jnp.float32),
        scratch_types=[
            pltpu.VMEM((CPW, CHUNK), jnp.int32),
            pltpu.VMEM((CHUNK, DG), jnp.float32),
            pltpu.VMEM((CHUNK, DG), jnp.float32),
            pltpu.VMEM_SHARED((NP, DG), jnp.float32),
            pltpu.SemaphoreType.DMA,
            pltpu.SemaphoreType.DMA,
            pltpu.SemaphoreType.DMA,
        ],
    )
    msg_k = pl.kernel(
        _msg_body,
        mesh=mesh,
        out_type=jax.ShapeDtypeStruct((NC, NP, D), jnp.float32),
        scratch_types=[
            pltpu.VMEM((IDXB, CHUNK), jnp.int32),
            pltpu.VMEM((IDXB, CHUNK), jnp.int32),
            pltpu.VMEM((CHUNK, D), jnp.float32),
            pltpu.VMEM((CHUNK, D), jnp.float32),
            pltpu.VMEM_SHARED((NP, D), jnp.float32),
            pltpu.SemaphoreType.DMA,
            pltpu.SemaphoreType.DMA,
        ],
    )
    return deg_k, msg_k


def kernel(x, edge_index, W1, b1, W2, b2, Wl, bl):
    _deg_kernel, _msg_kernel = _sc_kernels()
    src = edge_index[0]
    dst = edge_index[1]
    pad = jnp.full((EPAD - E,), N, jnp.int32)
    srcc = jnp.concatenate([src, pad]).reshape(NW, CPW, CHUNK)
    dstc = jnp.concatenate([dst, pad]).reshape(NW, CPW, CHUNK)

    const = jnp.stack([jnp.ones((CHUNK, DG), jnp.float32),
                       jnp.zeros((CHUNK, DG), jnp.float32)])
    xw = _tcmm(x, W1)
    degp = _deg_kernel(dstc, const)
    y1, dinv = _tc1(xw, degp)
    accp1 = _msg_kernel(y1, srcc, dstc)
    y2 = _tc2(accp1, y1, dinv, b1.reshape(1, D), W2)
    accp2 = _msg_kernel(y2, srcc, dstc)
    out = _tc3(accp2, y2, dinv, b2.reshape(1, D), Wl, bl.reshape(1, 1))
    return out
